# baseline (speedup 1.0000x reference)
import functools

import jax
import jax.numpy as jnp
from jax import lax
from jax.experimental import pallas as pl
from jax.experimental.pallas import tpu as pltpu
from jax.experimental.pallas import tpu_sc as plsc

_CH = 32


def _embed_lookup(xu2, xi2, tu, ti, *, batch, dim):
    info = plsc.get_sparse_core_info()
    n_workers = info.num_cores * info.num_subcores
    b_per_w = batch // n_workers
    n_ch = b_per_w // _CH

    mesh = plsc.VectorSubcoreMesh(core_axis_name="c", subcore_axis_name="s")

    @functools.partial(
        pl.kernel,
        mesh=mesh,
        out_type=(
            jax.ShapeDtypeStruct((batch, dim), jnp.float32),
            jax.ShapeDtypeStruct((batch, dim), jnp.float32),
        ),
        scratch_types=[
            pltpu.VMEM((b_per_w,), jnp.int32),
            pltpu.VMEM((b_per_w,), jnp.int32),
            pltpu.SemaphoreType.DMA,
        ],
    )
    def k(xu_hbm, xi_hbm, tu_hbm, ti_hbm, yu_hbm, yi_hbm,
          xu_v, xi_v, sem):
        wid = lax.axis_index("s") * info.num_cores + lax.axis_index("c")
        base = wid * b_per_w

        pltpu.async_copy(xu_hbm.at[wid], xu_v, sem).wait()
        pltpu.async_copy(xi_hbm.at[wid], xi_v, sem).wait()

        def make_body(t_hbm, y_hbm, x_v):
            def body(c, carry):
                off = c * _CH
                for g in range(_CH // 16):
                    vec = x_v[pl.ds(off + g * 16, 16)]
                    for l in range(16):
                        pltpu.async_copy(
                            t_hbm.at[pl.ds(vec[l], 1)],
                            y_hbm.at[pl.ds(base + off + g * 16 + l, 1)],
                            sem,
                        )
                for j in range(_CH):
                    pltpu.make_async_copy(
                        t_hbm.at[pl.ds(0, 1)],
                        y_hbm.at[pl.ds(base, 1)],
                        sem,
                    ).wait()
                return carry
            return body

        lax.fori_loop(0, n_ch, make_body(tu_hbm, yu_hbm, xu_v), 0)
        lax.fori_loop(0, n_ch, make_body(ti_hbm, yi_hbm, xi_v), 0)

    return k(xu2, xi2, tu, ti)


def kernel(x_user, x_item, table_user, table_item):
    batch = x_user.shape[0]
    dim = table_user.shape[1]
    info = plsc.get_sparse_core_info()
    n_workers = info.num_cores * info.num_subcores
    b_per_w = batch // n_workers

    xu2 = x_user.astype(jnp.int32).reshape(n_workers, b_per_w)
    xi2 = x_item.astype(jnp.int32).reshape(n_workers, b_per_w)
    return _embed_lookup(xu2, xi2, table_user, table_item,
                         batch=batch, dim=dim)


# speedup vs baseline: 2.0719x; 2.0719x over previous
import functools

import jax
import jax.numpy as jnp
from jax import lax
from jax.experimental import pallas as pl
from jax.experimental.pallas import tpu as pltpu
from jax.experimental.pallas import tpu_sc as plsc

_CH = 32


def _embed_lookup(xu2, xi2, tu, ti, *, batch, dim):
    info = plsc.get_sparse_core_info()
    n_workers = info.num_cores * info.num_subcores
    b_per_w = batch // n_workers
    n_ch = b_per_w // _CH

    mesh = plsc.VectorSubcoreMesh(core_axis_name="c", subcore_axis_name="s")

    @functools.partial(
        pl.kernel,
        mesh=mesh,
        out_type=(
            jax.ShapeDtypeStruct((batch, dim), jnp.float32),
            jax.ShapeDtypeStruct((batch, dim), jnp.float32),
        ),
        scratch_types=[
            pltpu.VMEM((b_per_w,), jnp.int32),
            pltpu.VMEM((b_per_w,), jnp.int32),
            pltpu.VMEM((b_per_w, dim), jnp.float32),
            pltpu.SemaphoreType.DMA,
        ],
    )
    def k(xu_hbm, xi_hbm, tu_hbm, ti_hbm, yu_hbm, yi_hbm,
          xu_v, xi_v, rows_v, sem):
        wid = lax.axis_index("s") * info.num_cores + lax.axis_index("c")
        base = wid * b_per_w

        pltpu.async_copy(xu_hbm.at[wid], xu_v, sem).wait()
        pltpu.async_copy(xi_hbm.at[wid], xi_v, sem).wait()

        def make_body(t_hbm, rows_v, x_v):
            def body(c, carry):
                off = c * _CH
                for g in range(_CH // 16):
                    vec = x_v[pl.ds(off + g * 16, 16)]
                    for l in range(16):
                        pltpu.async_copy(
                            t_hbm.at[pl.ds(vec[l], 1)],
                            rows_v.at[pl.ds(off + g * 16 + l, 1)],
                            sem,
                        )
                for j in range(_CH):
                    pltpu.make_async_copy(
                        t_hbm.at[pl.ds(0, 1)],
                        rows_v.at[pl.ds(0, 1)],
                        sem,
                    ).wait()
                return carry
            return body

        lax.fori_loop(0, n_ch, make_body(tu_hbm, rows_v, xu_v), 0)
        pltpu.async_copy(rows_v, yu_hbm.at[pl.ds(base, b_per_w)], sem).wait()
        lax.fori_loop(0, n_ch, make_body(ti_hbm, rows_v, xi_v), 0)
        pltpu.async_copy(rows_v, yi_hbm.at[pl.ds(base, b_per_w)], sem).wait()

    return k(xu2, xi2, tu, ti)


def kernel(x_user, x_item, table_user, table_item):
    batch = x_user.shape[0]
    dim = table_user.shape[1]
    info = plsc.get_sparse_core_info()
    n_workers = info.num_cores * info.num_subcores
    b_per_w = batch // n_workers

    xu2 = x_user.astype(jnp.int32).reshape(n_workers, b_per_w)
    xi2 = x_item.astype(jnp.int32).reshape(n_workers, b_per_w)
    return _embed_lookup(xu2, xi2, table_user, table_item,
                         batch=batch, dim=dim)


# speedup vs baseline: 2.1660x; 1.0454x over previous
import functools

import jax
import jax.numpy as jnp
from jax import lax
from jax.experimental import pallas as pl
from jax.experimental.pallas import tpu as pltpu
from jax.experimental.pallas import tpu_sc as plsc

_CH = 32


def _embed_lookup(xu2, xi2, tu, ti, *, batch, dim):
    info = plsc.get_sparse_core_info()
    n_workers = info.num_cores * info.num_subcores
    b_per_w = batch // n_workers
    n_ch = b_per_w // _CH

    mesh = plsc.VectorSubcoreMesh(core_axis_name="c", subcore_axis_name="s")

    @functools.partial(
        pl.kernel,
        mesh=mesh,
        out_type=(
            jax.ShapeDtypeStruct((batch, dim), jnp.float32),
            jax.ShapeDtypeStruct((batch, dim), jnp.float32),
        ),
        scratch_types=[
            pltpu.VMEM((b_per_w,), jnp.int32),
            pltpu.VMEM((b_per_w,), jnp.int32),
            pltpu.VMEM((b_per_w, dim), jnp.float32),
            pltpu.SemaphoreType.DMA,
        ],
    )
    def k(xu_hbm, xi_hbm, tu_hbm, ti_hbm, yu_hbm, yi_hbm,
          xu_v, xi_v, rows_v, sem):
        wid = lax.axis_index("s") * info.num_cores + lax.axis_index("c")
        base = wid * b_per_w

        pltpu.async_copy(xu_hbm.at[wid], xu_v, sem).wait()
        pltpu.async_copy(xi_hbm.at[wid], xi_v, sem).wait()

        def make_body(t_hbm, rows_v, x_v):
            def body(c, carry):
                off = c * _CH
                for g in range(_CH // 16):
                    vec = x_v[pl.ds(off + g * 16, 16)]
                    for l in range(16):
                        pltpu.async_copy(
                            t_hbm.at[pl.ds(vec[l], 1)],
                            rows_v.at[pl.ds(off + g * 16 + l, 1)],
                            sem,
                        )
                return carry
            return body

        def drain_and_store(y_hbm, rows_v):
            pltpu.make_async_copy(
                y_hbm.at[pl.ds(base, b_per_w)], rows_v, sem
            ).wait()
            pltpu.async_copy(
                rows_v, y_hbm.at[pl.ds(base, b_per_w)], sem
            ).wait()

        lax.fori_loop(0, n_ch, make_body(tu_hbm, rows_v, xu_v), 0)
        drain_and_store(yu_hbm, rows_v)
        lax.fori_loop(0, n_ch, make_body(ti_hbm, rows_v, xi_v), 0)
        drain_and_store(yi_hbm, rows_v)

    return k(xu2, xi2, tu, ti)


def kernel(x_user, x_item, table_user, table_item):
    batch = x_user.shape[0]
    dim = table_user.shape[1]
    info = plsc.get_sparse_core_info()
    n_workers = info.num_cores * info.num_subcores
    b_per_w = batch // n_workers

    xu2 = x_user.astype(jnp.int32).reshape(n_workers, b_per_w)
    xi2 = x_item.astype(jnp.int32).reshape(n_workers, b_per_w)
    return _embed_lookup(xu2, xi2, table_user, table_item,
                         batch=batch, dim=dim)
